# baseline (speedup 1.0000x reference)
import functools

import jax
import jax.numpy as jnp
from jax import lax
from jax.experimental import pallas as pl
from jax.experimental.pallas import tpu as pltpu
from jax.experimental.pallas import tpu_sc as plsc

MAX_POS = 8192
HIDDEN = 768
B = 4
L = 8192

N_TOK = B * L
LANES = 16
NC = 2
NS = 16
NW = NC * NS
F_TC = 8192
N_SC = N_TOK - F_TC
TPW = N_SC // NW
CHUNK = 16
NGROUP = TPW // CHUNK
NVREG = HIDDEN // LANES
SUBL = 6

_EPS = 1e-12
_INV_H = 1.0 / HIDDEN


def _rsqrt(u):
    yi = lax.bitcast_convert_type(u, jnp.int32)
    yi = jnp.full((LANES,), 0x5F3759DF, jnp.int32) - lax.shift_right_logical(
        yi, jnp.full((LANES,), 1, jnp.int32))
    g = lax.bitcast_convert_type(yi, jnp.float32)
    for _ in range(3):
        g = g * (1.5 - 0.5 * u * g * g)
    return g


_GATHER_DNUMS = lax.GatherDimensionNumbers(
    offset_dims=(), collapsed_slice_dims=(0,), start_index_map=(0,))


def _lane_sum(v):
    lanes = lax.iota(jnp.int32, LANES)
    for k in (8, 4, 2, 1):
        idx = lax.reshape(lanes ^ k, (LANES, 1))
        v = v + lax.gather(v, idx, _GATHER_DNUMS, (1,),
                           mode=lax.GatherScatterMode.PROMISE_IN_BOUNDS)
    return v


def _body(x_hbm, idx_hbm, t1_hbm, t2_hbm, w_hbm, b_hbm, out_hbm,
          idx_v, r1_v, r2_v, x_v, o_v, w_v, b_v,
          gsem0, gsem1, osem0, osem1):
    wid = lax.axis_index("s") * NC + lax.axis_index("c")
    base0 = wid * TPW
    gsem = (gsem0, gsem1)
    osem = (osem0, osem1)
    pltpu.sync_copy(w_hbm, w_v)
    pltpu.sync_copy(b_hbm, b_v)

    def fetch(g, b):
        base = base0 + g * CHUNK
        pltpu.sync_copy(idx_hbm.at[pl.ds(base, CHUNK)], idx_v.at[b])
        pltpu.async_copy(t1_hbm.at[idx_v.at[b]], r1_v.at[b], gsem[b])
        pltpu.async_copy(t2_hbm.at[idx_v.at[b]], r2_v.at[b], gsem[b])
        pltpu.async_copy(x_hbm.at[pl.ds(base, CHUNK)], x_v.at[b], gsem[b])

    def wait_fetch(g, b):
        base = base0 + g * CHUNK
        pltpu.make_async_copy(t1_hbm.at[idx_v.at[b]], r1_v.at[b], gsem[b]).wait()
        pltpu.make_async_copy(t2_hbm.at[idx_v.at[b]], r2_v.at[b], gsem[b]).wait()
        pltpu.make_async_copy(x_hbm.at[pl.ds(base, CHUNK)], x_v.at[b], gsem[b]).wait()

    def compute(b):
        zero = jnp.zeros((LANES,), jnp.float32)

        def pass1_pair(t0, t1):
            @plsc.parallel_loop(0, NVREG, step=2, unroll=2,
                                carry=(zero, zero, zero, zero,
                                       zero, zero, zero, zero))
            def p1(j, c):
                sa0, qa0, sa1, qa1, sb0, qb0, sb1, qb1 = c
                sl0 = pl.ds(j * LANES, LANES)
                sl1 = pl.ds((j + 1) * LANES, LANES)
                va0 = r1_v[b, t0, sl0] * x_v[b, t0, sl0] + r2_v[b, t0, sl0]
                va1 = r1_v[b, t0, sl1] * x_v[b, t0, sl1] + r2_v[b, t0, sl1]
                vb0 = r1_v[b, t1, sl0] * x_v[b, t1, sl0] + r2_v[b, t1, sl0]
                vb1 = r1_v[b, t1, sl1] * x_v[b, t1, sl1] + r2_v[b, t1, sl1]
                o_v[b, t0, sl0] = va0
                o_v[b, t0, sl1] = va1
                o_v[b, t1, sl0] = vb0
                o_v[b, t1, sl1] = vb1
                return (sa0 + va0, qa0 + va0 * va0, sa1 + va1, qa1 + va1 * va1,
                        sb0 + vb0, qb0 + vb0 * vb0, sb1 + vb1, qb1 + vb1 * vb1)

            sa0, qa0, sa1, qa1, sb0, qb0, sb1, qb1 = p1
            return sa0 + sa1, qa0 + qa1, sb0 + sb1, qb0 + qb1

        TGRP = 8

        @plsc.parallel_loop(0, CHUNK, step=TGRP)
        def token(t):
            means = []
            invs = []
            for k in range(0, TGRP, 2):
                sa, qa, sb, qb = pass1_pair(t + k, t + k + 1)
                for s, q in ((sa, qa), (sb, qb)):
                    m = _lane_sum(s) * _INV_H
                    means.append(m)
                    invs.append(_rsqrt(_lane_sum(q) * _INV_H - m * m + _EPS))

            @plsc.parallel_loop(0, NVREG, unroll=2)
            def pass2(j):
                sl = pl.ds(j * LANES, LANES)
                wv = w_v[sl]
                bv = b_v[sl]
                for k in range(TGRP):
                    o_v[b, t + k, sl] = (
                        (o_v[b, t + k, sl] - means[k]) * (wv * invs[k]) + bv)

    fetch(0, 0)

    def pair(p, carry):
        for b in (0, 1):
            g = p * 2 + b
            wait_fetch(g, b)

            @pl.when(g + 1 < NGROUP)
            def _():
                fetch(g + 1, 1 - b)

            base = base0 + g * CHUNK

            @pl.when(g >= 2)
            def _():
                pltpu.make_async_copy(
                    o_v.at[b], out_hbm.at[pl.ds(base, CHUNK)], osem[b]).wait()

            compute(b)
            pltpu.async_copy(o_v.at[b], out_hbm.at[pl.ds(base, CHUNK)], osem[b])
        return carry

    lax.fori_loop(0, NGROUP // 2, pair, 0)
    for b in (0, 1):
        base = base0 + (NGROUP - 2 + b) * CHUNK
        pltpu.make_async_copy(
            o_v.at[b], out_hbm.at[pl.ds(base, CHUNK)], osem[b]).wait()


@jax.jit
def _fc_embed(x2d, ids, t1, t2, w, b):
    mesh = plsc.VectorSubcoreMesh(core_axis_name="c", subcore_axis_name="s")
    f = functools.partial(
        pl.kernel,
        mesh=mesh,
        out_type=jax.ShapeDtypeStruct((N_SC, HIDDEN), jnp.float32),
        scratch_types=[
            pltpu.VMEM((2, CHUNK), jnp.int32),
            pltpu.VMEM((2, CHUNK, HIDDEN), jnp.float32),
            pltpu.VMEM((2, CHUNK, HIDDEN), jnp.float32),
            pltpu.VMEM((2, CHUNK, HIDDEN), jnp.float32),
            pltpu.VMEM((2, CHUNK, HIDDEN), jnp.float32),
            pltpu.VMEM((HIDDEN,), jnp.float32),
            pltpu.VMEM((HIDDEN,), jnp.float32),
            pltpu.SemaphoreType.DMA,
            pltpu.SemaphoreType.DMA,
            pltpu.SemaphoreType.DMA,
            pltpu.SemaphoreType.DMA,
        ],
    )(_body)
    return f(x2d, ids, t1, t2, w, b)


def _tc_body(ids_ref, x_ref, t1_ref, t2_ref, w_ref, b_ref, o_ref):
    i = pl.program_id(0)
    r = ids_ref[i] * SUBL
    v = t1_ref[pl.ds(r, SUBL), :] * x_ref[0] + t2_ref[pl.ds(r, SUBL), :]
    mean = jnp.mean(v)
    var = jnp.mean(v * v) - mean * mean
    inv = lax.rsqrt(var + _EPS)
    o_ref[0] = (v - mean) * inv * w_ref[...] + b_ref[...]


@jax.jit
def _tc_embed(x3d, ids, t1_3d, t2_3d, w2d, b2d):
    grid_spec = pltpu.PrefetchScalarGridSpec(
        num_scalar_prefetch=1,
        grid=(F_TC,),
        in_specs=[
            pl.BlockSpec((1, SUBL, 128), lambda i, ids: (i, 0, 0)),
            pl.BlockSpec((MAX_POS * SUBL, 128), lambda i, ids: (0, 0)),
            pl.BlockSpec((MAX_POS * SUBL, 128), lambda i, ids: (0, 0)),
            pl.BlockSpec((SUBL, 128), lambda i, ids: (0, 0)),
            pl.BlockSpec((SUBL, 128), lambda i, ids: (0, 0)),
        ],
        out_specs=pl.BlockSpec((1, SUBL, 128), lambda i, ids: (i, 0, 0)),
    )
    return pl.pallas_call(
        _tc_body,
        grid_spec=grid_spec,
        out_shape=jax.ShapeDtypeStruct((F_TC, SUBL, 128), jnp.float32),
        compiler_params=pltpu.CompilerParams(
            vmem_limit_bytes=110 * 1024 * 1024),
    )(ids, x3d, t1_3d, t2_3d, w2d, b2d)


def kernel(inputs_embeds, position_ids, pos_table1, pos_table2, ln_weight, ln_bias):
    x2d = inputs_embeds.reshape(N_TOK, HIDDEN)
    ids = position_ids.reshape(N_TOK).astype(jnp.int32)
    w2d = ln_weight.reshape(SUBL, 128)
    b2d = ln_bias.reshape(SUBL, 128)
    out_tc = _tc_embed(
        x2d[:F_TC].reshape(F_TC, SUBL, 128), ids[:F_TC],
        pos_table1.reshape(MAX_POS * SUBL, 128),
        pos_table2.reshape(MAX_POS * SUBL, 128), w2d, b2d)
    out_sc = _fc_embed(x2d[F_TC:], ids[F_TC:], pos_table1, pos_table2,
                       ln_weight, ln_bias)
    out = jnp.concatenate(
        [out_tc.reshape(F_TC, HIDDEN), out_sc], axis=0)
    return out.reshape(B, L, HIDDEN)


# speedup vs baseline: 5.3285x; 5.3285x over previous
import functools

import jax
import jax.numpy as jnp
from jax import lax
from jax.experimental import pallas as pl
from jax.experimental.pallas import tpu as pltpu
from jax.experimental.pallas import tpu_sc as plsc

MAX_POS = 8192
HIDDEN = 768
B = 4
L = 8192

N_TOK = B * L
LANES = 16
NC = 2
NS = 16
NW = NC * NS
F_TC = 8192
N_SC = N_TOK - F_TC
TPW = N_SC // NW
CHUNK = 16
NGROUP = TPW // CHUNK
NVREG = HIDDEN // LANES
SUBL = 6

_EPS = 1e-12
_INV_H = 1.0 / HIDDEN


def _rsqrt(u):
    yi = lax.bitcast_convert_type(u, jnp.int32)
    yi = jnp.full((LANES,), 0x5F3759DF, jnp.int32) - lax.shift_right_logical(
        yi, jnp.full((LANES,), 1, jnp.int32))
    g = lax.bitcast_convert_type(yi, jnp.float32)
    for _ in range(3):
        g = g * (1.5 - 0.5 * u * g * g)
    return g


_GATHER_DNUMS = lax.GatherDimensionNumbers(
    offset_dims=(), collapsed_slice_dims=(0,), start_index_map=(0,))


def _lane_sum(v):
    lanes = lax.iota(jnp.int32, LANES)
    for k in (8, 4, 2, 1):
        idx = lax.reshape(lanes ^ k, (LANES, 1))
        v = v + lax.gather(v, idx, _GATHER_DNUMS, (1,),
                           mode=lax.GatherScatterMode.PROMISE_IN_BOUNDS)
    return v


def _body(x_hbm, idx_hbm, t1_hbm, t2_hbm, w_hbm, b_hbm, out_hbm,
          idx_v, r1_v, r2_v, x_v, o_v, w_v, b_v,
          gsem0, gsem1, osem0, osem1):
    wid = lax.axis_index("s") * NC + lax.axis_index("c")
    base0 = wid * TPW
    gsem = (gsem0, gsem1)
    osem = (osem0, osem1)
    pltpu.sync_copy(w_hbm, w_v)
    pltpu.sync_copy(b_hbm, b_v)

    def fetch(g, b):
        base = base0 + g * CHUNK
        pltpu.sync_copy(idx_hbm.at[pl.ds(base, CHUNK)], idx_v.at[b])
        pltpu.async_copy(t1_hbm.at[idx_v.at[b]], r1_v.at[b], gsem[b])
        pltpu.async_copy(t2_hbm.at[idx_v.at[b]], r2_v.at[b], gsem[b])
        pltpu.async_copy(x_hbm.at[pl.ds(base, CHUNK)], x_v.at[b], gsem[b])

    def wait_fetch(g, b):
        base = base0 + g * CHUNK
        pltpu.make_async_copy(t1_hbm.at[idx_v.at[b]], r1_v.at[b], gsem[b]).wait()
        pltpu.make_async_copy(t2_hbm.at[idx_v.at[b]], r2_v.at[b], gsem[b]).wait()
        pltpu.make_async_copy(x_hbm.at[pl.ds(base, CHUNK)], x_v.at[b], gsem[b]).wait()

    def compute(b):
        zero = jnp.zeros((LANES,), jnp.float32)

        def pass1_pair(t0, t1):
            @plsc.parallel_loop(0, NVREG, step=2, unroll=2,
                                carry=(zero, zero, zero, zero,
                                       zero, zero, zero, zero))
            def p1(j, c):
                sa0, qa0, sa1, qa1, sb0, qb0, sb1, qb1 = c
                sl0 = pl.ds(j * LANES, LANES)
                sl1 = pl.ds((j + 1) * LANES, LANES)
                va0 = r1_v[b, t0, sl0] * x_v[b, t0, sl0] + r2_v[b, t0, sl0]
                va1 = r1_v[b, t0, sl1] * x_v[b, t0, sl1] + r2_v[b, t0, sl1]
                vb0 = r1_v[b, t1, sl0] * x_v[b, t1, sl0] + r2_v[b, t1, sl0]
                vb1 = r1_v[b, t1, sl1] * x_v[b, t1, sl1] + r2_v[b, t1, sl1]
                o_v[b, t0, sl0] = va0
                o_v[b, t0, sl1] = va1
                o_v[b, t1, sl0] = vb0
                o_v[b, t1, sl1] = vb1
                return (sa0 + va0, qa0 + va0 * va0, sa1 + va1, qa1 + va1 * va1,
                        sb0 + vb0, qb0 + vb0 * vb0, sb1 + vb1, qb1 + vb1 * vb1)

            sa0, qa0, sa1, qa1, sb0, qb0, sb1, qb1 = p1
            return sa0 + sa1, qa0 + qa1, sb0 + sb1, qb0 + qb1

        TGRP = 8

        @plsc.parallel_loop(0, CHUNK, step=TGRP)
        def token(t):
            means = []
            invs = []
            for k in range(0, TGRP, 2):
                sa, qa, sb, qb = pass1_pair(t + k, t + k + 1)
                for s, q in ((sa, qa), (sb, qb)):
                    m = _lane_sum(s) * _INV_H
                    means.append(m)
                    invs.append(_rsqrt(_lane_sum(q) * _INV_H - m * m + _EPS))

            @plsc.parallel_loop(0, NVREG, unroll=2)
            def pass2(j):
                sl = pl.ds(j * LANES, LANES)
                wv = w_v[sl]
                bv = b_v[sl]
                for k in range(TGRP):
                    o_v[b, t + k, sl] = (
                        (o_v[b, t + k, sl] - means[k]) * (wv * invs[k]) + bv)

    fetch(0, 0)

    def pair(p, carry):
        for b in (0, 1):
            g = p * 2 + b
            wait_fetch(g, b)

            @pl.when(g + 1 < NGROUP)
            def _():
                fetch(g + 1, 1 - b)

            base = base0 + g * CHUNK

            @pl.when(g >= 2)
            def _():
                pltpu.make_async_copy(
                    o_v.at[b], out_hbm.at[pl.ds(base, CHUNK)], osem[b]).wait()

            compute(b)
            pltpu.async_copy(o_v.at[b], out_hbm.at[pl.ds(base, CHUNK)], osem[b])
        return carry

    lax.fori_loop(0, NGROUP // 2, pair, 0)
    for b in (0, 1):
        base = base0 + (NGROUP - 2 + b) * CHUNK
        pltpu.make_async_copy(
            o_v.at[b], out_hbm.at[pl.ds(base, CHUNK)], osem[b]).wait()


@jax.jit
def _fc_embed(x2d, ids, t1, t2, w, b):
    mesh = plsc.VectorSubcoreMesh(core_axis_name="c", subcore_axis_name="s")
    f = functools.partial(
        pl.kernel,
        mesh=mesh,
        out_type=jax.ShapeDtypeStruct((N_SC, HIDDEN), jnp.float32),
        scratch_types=[
            pltpu.VMEM((2, CHUNK), jnp.int32),
            pltpu.VMEM((2, CHUNK, HIDDEN), jnp.float32),
            pltpu.VMEM((2, CHUNK, HIDDEN), jnp.float32),
            pltpu.VMEM((2, CHUNK, HIDDEN), jnp.float32),
            pltpu.VMEM((2, CHUNK, HIDDEN), jnp.float32),
            pltpu.VMEM((HIDDEN,), jnp.float32),
            pltpu.VMEM((HIDDEN,), jnp.float32),
            pltpu.SemaphoreType.DMA,
            pltpu.SemaphoreType.DMA,
            pltpu.SemaphoreType.DMA,
            pltpu.SemaphoreType.DMA,
        ],
    )(_body)
    return f(x2d, ids, t1, t2, w, b)


TB_TC = 32


def _tc_body(ids_ref, x_ref, t1_ref, t2_ref, w_ref, b_ref, o_ref):
    i = pl.program_id(0)
    wv = w_ref[...]
    bv = b_ref[...]
    for k in range(TB_TC):
        r = ids_ref[i * TB_TC + k] * SUBL
        v = t1_ref[pl.ds(r, SUBL), :] * x_ref[k] + t2_ref[pl.ds(r, SUBL), :]
        mean = jnp.mean(v)
        var = jnp.mean(v * v) - mean * mean
        inv = lax.rsqrt(var + _EPS)
        o_ref[k] = (v - mean) * inv * wv + bv


@jax.jit
def _tc_embed(x3d, ids, t1_3d, t2_3d, w2d, b2d):
    grid_spec = pltpu.PrefetchScalarGridSpec(
        num_scalar_prefetch=1,
        grid=(F_TC // TB_TC,),
        in_specs=[
            pl.BlockSpec((TB_TC, SUBL, 128), lambda i, ids: (i, 0, 0)),
            pl.BlockSpec((MAX_POS * SUBL, 128), lambda i, ids: (0, 0)),
            pl.BlockSpec((MAX_POS * SUBL, 128), lambda i, ids: (0, 0)),
            pl.BlockSpec((SUBL, 128), lambda i, ids: (0, 0)),
            pl.BlockSpec((SUBL, 128), lambda i, ids: (0, 0)),
        ],
        out_specs=pl.BlockSpec((TB_TC, SUBL, 128), lambda i, ids: (i, 0, 0)),
    )
    return pl.pallas_call(
        _tc_body,
        grid_spec=grid_spec,
        out_shape=jax.ShapeDtypeStruct((F_TC, SUBL, 128), jnp.float32),
        compiler_params=pltpu.CompilerParams(
            vmem_limit_bytes=110 * 1024 * 1024),
    )(ids, x3d, t1_3d, t2_3d, w2d, b2d)


def kernel(inputs_embeds, position_ids, pos_table1, pos_table2, ln_weight, ln_bias):
    x2d = inputs_embeds.reshape(N_TOK, HIDDEN)
    ids = position_ids.reshape(N_TOK).astype(jnp.int32)
    w2d = ln_weight.reshape(SUBL, 128)
    b2d = ln_bias.reshape(SUBL, 128)
    out_tc = _tc_embed(
        x2d[:F_TC].reshape(F_TC, SUBL, 128), ids[:F_TC],
        pos_table1.reshape(MAX_POS * SUBL, 128),
        pos_table2.reshape(MAX_POS * SUBL, 128), w2d, b2d)
    out_sc = _fc_embed(x2d[F_TC:], ids[F_TC:], pos_table1, pos_table2,
                       ln_weight, ln_bias)
    out = jnp.concatenate(
        [out_tc.reshape(F_TC, HIDDEN), out_sc], axis=0)
    return out.reshape(B, L, HIDDEN)


# speedup vs baseline: 21.7742x; 4.0864x over previous
import functools

import jax
import jax.numpy as jnp
from jax import lax
from jax.experimental import pallas as pl
from jax.experimental.pallas import tpu as pltpu
from jax.experimental.pallas import tpu_sc as plsc

MAX_POS = 8192
HIDDEN = 768
B = 4
L = 8192

N_TOK = B * L
LANES = 16
NC = 2
NS = 16
NW = NC * NS
N_SC = N_TOK
TPW = N_SC // NW
CHUNK = 16
NGROUP = TPW // CHUNK
NVREG = HIDDEN // LANES

_EPS = 1e-12
_INV_H = 1.0 / HIDDEN


def _rsqrt(u):
    yi = lax.bitcast_convert_type(u, jnp.int32)
    yi = jnp.full((LANES,), 0x5F3759DF, jnp.int32) - lax.shift_right_logical(
        yi, jnp.full((LANES,), 1, jnp.int32))
    g = lax.bitcast_convert_type(yi, jnp.float32)
    for _ in range(3):
        g = g * (1.5 - 0.5 * u * g * g)
    return g


_GATHER_DNUMS = lax.GatherDimensionNumbers(
    offset_dims=(), collapsed_slice_dims=(0,), start_index_map=(0,))


def _lane_sum(v):
    lanes = lax.iota(jnp.int32, LANES)
    for k in (8, 4, 2, 1):
        idx = lax.reshape(lanes ^ k, (LANES, 1))
        v = v + lax.gather(v, idx, _GATHER_DNUMS, (1,),
                           mode=lax.GatherScatterMode.PROMISE_IN_BOUNDS)
    return v


def _body(x_hbm, idx_hbm, t1_hbm, t2_hbm, w_hbm, b_hbm, out_hbm,
          idx_v, r1_v, r2_v, x_v, o_v, w_v, b_v,
          gsem0, gsem1, osem0, osem1):
    wid = lax.axis_index("s") * NC + lax.axis_index("c")
    base0 = wid * TPW
    gsem = (gsem0, gsem1)
    osem = (osem0, osem1)
    pltpu.sync_copy(w_hbm, w_v)
    pltpu.sync_copy(b_hbm, b_v)

    def fetch(g, b):
        base = base0 + g * CHUNK
        pltpu.sync_copy(idx_hbm.at[pl.ds(base, CHUNK)], idx_v.at[b])
        pltpu.async_copy(t1_hbm.at[idx_v.at[b]], r1_v.at[b], gsem[b])
        pltpu.async_copy(t2_hbm.at[idx_v.at[b]], r2_v.at[b], gsem[b])
        pltpu.async_copy(x_hbm.at[pl.ds(base, CHUNK)], x_v.at[b], gsem[b])

    def wait_fetch(g, b):
        base = base0 + g * CHUNK
        pltpu.make_async_copy(t1_hbm.at[idx_v.at[b]], r1_v.at[b], gsem[b]).wait()
        pltpu.make_async_copy(t2_hbm.at[idx_v.at[b]], r2_v.at[b], gsem[b]).wait()
        pltpu.make_async_copy(x_hbm.at[pl.ds(base, CHUNK)], x_v.at[b], gsem[b]).wait()

    def compute(b):
        zero = jnp.zeros((LANES,), jnp.float32)

        def pass1_pair(t0, t1):
            @plsc.parallel_loop(0, NVREG, step=2, unroll=2,
                                carry=(zero, zero, zero, zero,
                                       zero, zero, zero, zero))
            def p1(j, c):
                sa0, qa0, sa1, qa1, sb0, qb0, sb1, qb1 = c
                sl0 = pl.ds(j * LANES, LANES)
                sl1 = pl.ds((j + 1) * LANES, LANES)
                va0 = r1_v[b, t0, sl0] * x_v[b, t0, sl0] + r2_v[b, t0, sl0]
                va1 = r1_v[b, t0, sl1] * x_v[b, t0, sl1] + r2_v[b, t0, sl1]
                vb0 = r1_v[b, t1, sl0] * x_v[b, t1, sl0] + r2_v[b, t1, sl0]
                vb1 = r1_v[b, t1, sl1] * x_v[b, t1, sl1] + r2_v[b, t1, sl1]
                o_v[b, t0, sl0] = va0
                o_v[b, t0, sl1] = va1
                o_v[b, t1, sl0] = vb0
                o_v[b, t1, sl1] = vb1
                return (sa0 + va0, qa0 + va0 * va0, sa1 + va1, qa1 + va1 * va1,
                        sb0 + vb0, qb0 + vb0 * vb0, sb1 + vb1, qb1 + vb1 * vb1)

            sa0, qa0, sa1, qa1, sb0, qb0, sb1, qb1 = p1
            return sa0 + sa1, qa0 + qa1, sb0 + sb1, qb0 + qb1

        TGRP = 8

        @plsc.parallel_loop(0, CHUNK, step=TGRP)
        def token(t):
            means = []
            invs = []
            for k in range(0, TGRP, 2):
                sa, qa, sb, qb = pass1_pair(t + k, t + k + 1)
                for s, q in ((sa, qa), (sb, qb)):
                    m = _lane_sum(s) * _INV_H
                    means.append(m)
                    invs.append(_rsqrt(_lane_sum(q) * _INV_H - m * m + _EPS))

            @plsc.parallel_loop(0, NVREG, unroll=2)
            def pass2(j):
                sl = pl.ds(j * LANES, LANES)
                wv = w_v[sl]
                bv = b_v[sl]
                for k in range(TGRP):
                    o_v[b, t + k, sl] = (
                        (o_v[b, t + k, sl] - means[k]) * (wv * invs[k]) + bv)

    fetch(0, 0)

    def pair(p, carry):
        for b in (0, 1):
            g = p * 2 + b
            wait_fetch(g, b)

            @pl.when(g + 1 < NGROUP)
            def _():
                fetch(g + 1, 1 - b)

            base = base0 + g * CHUNK

            @pl.when(g >= 2)
            def _():
                pltpu.make_async_copy(
                    o_v.at[b], out_hbm.at[pl.ds(base, CHUNK)], osem[b]).wait()

            compute(b)
            pltpu.async_copy(o_v.at[b], out_hbm.at[pl.ds(base, CHUNK)], osem[b])
        return carry

    lax.fori_loop(0, NGROUP // 2, pair, 0)
    for b in (0, 1):
        base = base0 + (NGROUP - 2 + b) * CHUNK
        pltpu.make_async_copy(
            o_v.at[b], out_hbm.at[pl.ds(base, CHUNK)], osem[b]).wait()


@jax.jit
def _fc_embed(x2d, ids, t1, t2, w, b):
    mesh = plsc.VectorSubcoreMesh(core_axis_name="c", subcore_axis_name="s")
    f = functools.partial(
        pl.kernel,
        mesh=mesh,
        out_type=jax.ShapeDtypeStruct((N_SC, HIDDEN), jnp.float32),
        scratch_types=[
            pltpu.VMEM((2, CHUNK), jnp.int32),
            pltpu.VMEM((2, CHUNK, HIDDEN), jnp.float32),
            pltpu.VMEM((2, CHUNK, HIDDEN), jnp.float32),
            pltpu.VMEM((2, CHUNK, HIDDEN), jnp.float32),
            pltpu.VMEM((2, CHUNK, HIDDEN), jnp.float32),
            pltpu.VMEM((HIDDEN,), jnp.float32),
            pltpu.VMEM((HIDDEN,), jnp.float32),
            pltpu.SemaphoreType.DMA,
            pltpu.SemaphoreType.DMA,
            pltpu.SemaphoreType.DMA,
            pltpu.SemaphoreType.DMA,
        ],
    )(_body)
    return f(x2d, ids, t1, t2, w, b)


def kernel(inputs_embeds, position_ids, pos_table1, pos_table2, ln_weight, ln_bias):
    x2d = inputs_embeds.reshape(N_TOK, HIDDEN)
    ids = position_ids.reshape(N_TOK).astype(jnp.int32)
    out = _fc_embed(x2d, ids, pos_table1, pos_table2, ln_weight, ln_bias)
    return out.reshape(B, L, HIDDEN)
